# baseline (speedup 1.0000x reference)
import functools

import jax
import jax.numpy as jnp
from jax import lax
from jax.experimental import pallas as pl
from jax.experimental.pallas import tpu as pltpu
from jax.experimental.pallas import tpu_sc as plsc

N_NODES = 10000
N_EDGES = 320000
N_GRAPHS = 256
NT = 16
NC = 2
CHUNK = 128
CH = 64
SLOTS = 4
NITER = 79
EPT = NITER * SLOTS * CH
PAD = EPT * NT - N_EDGES
S_ROWS = 10240
TRASH = N_NODES
HW = 128


def _sc_mesh():
    return plsc.VectorSubcoreMesh(core_axis_name="c", subcore_axis_name="s",
                                  num_cores=NC, num_subcores=NT)


@functools.cache
def _make_degree():
    @functools.partial(
        pl.kernel,
        out_type=jax.ShapeDtypeStruct((NC, S_ROWS, HW), jnp.float32),
        mesh=_sc_mesh(),
        scratch_types=[
            pltpu.VMEM((2, SLOTS, 2, CH), jnp.int32),
            pltpu.VMEM((CH, HW), jnp.float32),
            pltpu.VMEM_SHARED((S_ROWS, HW), jnp.float32),
            pltpu.SemaphoreType.DMA,
        ],
    )
    def degree(edg_hbm, ones_hbm, zeros_hbm, out_hbm, dv, ones_v, hist_sh,
               isem):
        c = lax.axis_index("c")
        s = lax.axis_index("s")
        pltpu.sync_copy(ones_hbm, ones_v)
        pltpu.sync_copy(zeros_hbm,
                        hist_sh.at[pl.ds(s * (S_ROWS // NT), S_ROWS // NT)])
        plsc.subcore_barrier()

        def iwait(dst):
            pltpu.make_async_copy(edg_hbm.at[0, 0, 0, 0], dst, isem).wait()

        for k in range(SLOTS):
            pltpu.async_copy(edg_hbm.at[c, s, 0, k], dv.at[0, k], isem)
        for k in range(SLOTS):
            iwait(dv.at[0, k])
        for k in range(SLOTS):
            pltpu.async_copy(edg_hbm.at[c, s, 1, k], dv.at[1, k], isem)

        def body(t, carry):
            bank = lax.rem(t, 2)
            nbank = 1 - bank
            t2 = jnp.minimum(t + 2, NITER - 1)
            for k in range(SLOTS):
                pltpu.sync_copy(ones_v, hist_sh.at[dv.at[bank, k, 1]],
                                add=True)
            for k in range(SLOTS):
                iwait(dv.at[nbank, k])
            for k in range(SLOTS):
                pltpu.async_copy(edg_hbm.at[c, s, t2, k], dv.at[bank, k],
                                 isem)
            return carry

        lax.fori_loop(0, NITER, body, 0)
        for k in range(SLOTS):
            iwait(dv.at[0, k])
        plsc.subcore_barrier()
        rows = S_ROWS // NT
        pltpu.sync_copy(hist_sh.at[pl.ds(s * rows, rows)],
                        out_hbm.at[c, pl.ds(s * rows, rows)])

    return degree


def _sc_degree(edges, ones_h, zeros_h):
    return _make_degree()(edges, ones_h, zeros_h)


@functools.cache
def _make_msgpass(d):
    @functools.partial(
        pl.kernel,
        out_type=jax.ShapeDtypeStruct((NC * S_ROWS, d), jnp.float32),
        mesh=_sc_mesh(),
        scratch_types=[
            pltpu.VMEM((2, SLOTS, 2, CH), jnp.int32),
            pltpu.VMEM((SLOTS, CH, d), jnp.float32),
            pltpu.VMEM_SHARED((S_ROWS, d), jnp.float32),
            pltpu.SemaphoreType.DMA,
            pltpu.SemaphoreType.DMA,
            pltpu.SemaphoreType.DMA,
        ],
    )
    def msgpass(g_hbm, edg_hbm, zeros_hbm, out_hbm,
                idxv, rows, s_sh, gsem, ssem, isem):
        c = lax.axis_index("c")
        s = lax.axis_index("s")
        stripe = S_ROWS // NT
        pltpu.sync_copy(zeros_hbm, s_sh.at[pl.ds(s * stripe, stripe)])
        plsc.subcore_barrier()

        def wait(sem, dst):
            pltpu.make_async_copy(g_hbm.at[pl.ds(0, CH)]
                                  if dst.dtype == jnp.float32
                                  else edg_hbm.at[0, 0, 0, 0], dst, sem).wait()

        for k in range(SLOTS):
            pltpu.async_copy(edg_hbm.at[c, s, 0, k], idxv.at[0, k], isem)
        for k in range(SLOTS):
            wait(isem, idxv.at[0, k])
        for k in range(SLOTS):
            pltpu.async_copy(g_hbm.at[idxv.at[0, k, 0]], rows.at[k], gsem)
        for k in range(SLOTS):
            pltpu.async_copy(edg_hbm.at[c, s, 1, k], idxv.at[1, k], isem)

        def body(t, carry):
            bank = lax.rem(t, 2)
            nbank = 1 - bank
            t2 = jnp.minimum(t + 2, NITER - 1)
            for k in range(SLOTS):
                wait(gsem, rows.at[k])
            for k in range(SLOTS):
                pltpu.async_copy(rows.at[k], s_sh.at[idxv.at[bank, k, 1]],
                                 ssem, add=True)
            for k in range(SLOTS):
                wait(isem, idxv.at[nbank, k])
            for k in range(SLOTS):
                wait(ssem, rows.at[k])
            for k in range(SLOTS):
                pltpu.async_copy(g_hbm.at[idxv.at[nbank, k, 0]], rows.at[k],
                                 gsem)
            for k in range(SLOTS):
                pltpu.async_copy(edg_hbm.at[c, s, t2, k], idxv.at[bank, k],
                                 isem)
            return carry

        lax.fori_loop(0, NITER, body, 0)
        for k in range(SLOTS):
            wait(gsem, rows.at[k])
            wait(isem, idxv.at[0, k])
        plsc.subcore_barrier()
        pltpu.sync_copy(s_sh.at[pl.ds(s * stripe, stripe)],
                        out_hbm.at[pl.ds(c * S_ROWS + s * stripe, stripe)])

    return msgpass


def _msgpass128(g, edges, zeros):
    return _make_msgpass(128)(g, edges, zeros)


_BM = 1024


def _tc_layer1(x, deg, w):
    def body(x_ref, deg_ref, w_ref, o_ref):
        dinv = lax.rsqrt(deg_ref[:, 0:1] + 1.0)
        o_ref[...] = dinv * jnp.dot(x_ref[...], w_ref[...],
                                    preferred_element_type=jnp.float32)

    m = NC * S_ROWS
    return pl.pallas_call(
        body,
        grid=(m // _BM,),
        in_specs=[
            pl.BlockSpec((_BM, 128), lambda i: (i, 0)),
            pl.BlockSpec((_BM, HW), lambda i: (i, 0)),
            pl.BlockSpec((128, 128), lambda i: (0, 0)),
        ],
        out_specs=pl.BlockSpec((_BM, 128), lambda i: (i, 0)),
        out_shape=jax.ShapeDtypeStruct((m, 128), jnp.float32),
    )(x, deg, w)


def _tc_layer(s_in, g_prev, b_prev, deg, w, d_out):
    def body(s_ref, g_ref, deg_ref, b_ref, w_ref, o_ref):
        dinv = lax.rsqrt(deg_ref[:, 0:1] + 1.0)
        x = jax.nn.relu(dinv * (s_ref[...] + g_ref[...]) + b_ref[0:1, :])
        o_ref[...] = dinv * jnp.dot(x, w_ref[...],
                                    preferred_element_type=jnp.float32)

    m, d_in = s_in.shape
    return pl.pallas_call(
        body,
        grid=(m // _BM,),
        in_specs=[
            pl.BlockSpec((_BM, d_in), lambda i: (i, 0)),
            pl.BlockSpec((_BM, d_in), lambda i: (i, 0)),
            pl.BlockSpec((_BM, HW), lambda i: (i, 0)),
            pl.BlockSpec((8, d_in), lambda i: (0, 0)),
            pl.BlockSpec((d_in, d_out), lambda i: (0, 0)),
        ],
        out_specs=pl.BlockSpec((_BM, d_out), lambda i: (i, 0)),
        out_shape=jax.ShapeDtypeStruct((m, d_out), jnp.float32),
    )(s_in, g_prev, deg, b_prev, w)


_PB = 2048


def _tc_pool(s4, g4, b4, deg, ids3):
    nb = S_ROWS // _PB

    def body(s_ref, g_ref, deg_ref, b_ref, ids_ref, o_ref, cnt_ref):
        j = pl.program_id(1)
        dinv = lax.rsqrt(deg_ref[:, 0:1] + 1.0)
        x = jax.nn.relu(dinv * (s_ref[...] + g_ref[...]) + b_ref[0:1, :])[:, :64]
        ids = ids_ref[0, 0, :]
        gid = lax.broadcasted_iota(jnp.int32, (N_GRAPHS, _PB), 0)
        p = (gid == ids[None, :]).astype(jnp.float32)
        ps = jnp.dot(p, x, preferred_element_type=jnp.float32)
        pc = jnp.sum(p, axis=1, keepdims=True)

        @pl.when(j == 0)
        def _():
            o_ref[0] = ps
            cnt_ref[...] = jnp.broadcast_to(pc, (N_GRAPHS, 128))

        @pl.when(j != 0)
        def _():
            o_ref[0] += ps
            cnt_ref[...] += jnp.broadcast_to(pc, (N_GRAPHS, 128))

        @pl.when(j == nb - 1)
        def _():
            o_ref[0] /= jnp.maximum(cnt_ref[:, 0:1], 1.0)

    return pl.pallas_call(
        body,
        grid=(NC, nb),
        in_specs=[
            pl.BlockSpec((_PB, 128), lambda m, j: (m * nb + j, 0)),
            pl.BlockSpec((_PB, 128), lambda m, j: (m * nb + j, 0)),
            pl.BlockSpec((_PB, HW), lambda m, j: (m * nb + j, 0)),
            pl.BlockSpec((8, 128), lambda m, j: (0, 0)),
            pl.BlockSpec((1, 1, _PB), lambda m, j: (m * nb + j, 0, 0)),
        ],
        out_specs=pl.BlockSpec((1, N_GRAPHS, 64), lambda m, j: (m, 0, 0)),
        out_shape=jax.ShapeDtypeStruct((NC, N_GRAPHS, 64), jnp.float32),
        scratch_shapes=[pltpu.VMEM((N_GRAPHS, 128), jnp.float32)],
    )(s4, g4, deg, b4, ids3)


def _tc_classifier(pooled, n1, n2, fc1_w, fc1_b, fc2_w, fc2_b,
                   cls1_w, cls1_b, bn1_g, bn1_b,
                   cls2_w, cls2_b, bn2_g, bn2_b, cls3_w, cls3_b):
    def bn(h, g_ref, b_ref):
        mu = jnp.mean(h, axis=0, keepdims=True)
        var = jnp.mean((h - mu) ** 2, axis=0, keepdims=True)
        return (h - mu) * lax.rsqrt(var + 1e-5) * g_ref[0:1, :] + b_ref[0:1, :]

    def body(pooled_ref, n1_ref, n2_ref, fc1w_ref, fc1b_ref, fc2w_ref,
             fc2b_ref, c1w_ref, c1b_ref, g1_ref, b1_ref, c2w_ref, c2b_ref,
             g2_ref, b2_ref, c3w_ref, c3b_ref, o_ref):
        def notes_mlp(r):
            t = jax.nn.relu(jnp.dot(r, fc1w_ref[...],
                                    preferred_element_type=jnp.float32)
                            + fc1b_ref[0:1, :])
            return jax.nn.relu(jnp.dot(t, fc2w_ref[...],
                                       preferred_element_type=jnp.float32)
                               + fc2b_ref[0:1, :])

        m1 = notes_mlp(n1_ref[...])
        m2 = notes_mlp(n2_ref[...])
        comb = jnp.concatenate(
            [pooled_ref[0], pooled_ref[1], m1, m2], axis=1)
        h = jax.nn.relu(jnp.dot(comb, c1w_ref[...],
                                preferred_element_type=jnp.float32)
                        + c1b_ref[0:1, :])
        h = bn(h, g1_ref, b1_ref)
        h = jax.nn.relu(jnp.dot(h, c2w_ref[...],
                                preferred_element_type=jnp.float32)
                        + c2b_ref[0:1, :])
        h = bn(h, g2_ref, b2_ref)
        o_ref[...] = (jnp.dot(h, c3w_ref[...],
                              preferred_element_type=jnp.float32)
                      + c3b_ref[0:1, :])

    return pl.pallas_call(
        body,
        out_shape=jax.ShapeDtypeStruct((N_GRAPHS, 128), jnp.float32),
    )(pooled, n1, n2, fc1_w, fc1_b, fc2_w, fc2_b, cls1_w, cls1_b,
      bn1_g, bn1_b, cls2_w, cls2_b, bn2_g, bn2_b, cls3_w, cls3_b)


def _pad_edges(a, fill):
    a = jnp.concatenate([a, jnp.full((PAD,), fill, jnp.int32)])
    return a.reshape(NT, NITER, SLOTS, 1, CH)


def _tile8(b):
    return jnp.tile(b[None, :], (8, 1))


def kernel(mol1_x, mol1_edge_index, mol1_batch_ids, mol1_notes,
           mol2_x, mol2_edge_index, mol2_batch_ids, mol2_notes,
           conv1_W, conv1_b, conv2_W, conv2_b, conv3_W, conv3_b,
           conv4_W, conv4_b, fc1_W, fc1_b, fc2_W, fc2_b,
           cls1_W, cls1_b, bn1_g, bn1_b, cls2_W, cls2_b, bn2_g, bn2_b,
           cls3_W, cls3_b):
    i32 = jnp.int32
    src1 = mol1_edge_index[0].astype(i32)
    dst1 = mol1_edge_index[1].astype(i32)
    src2 = mol2_edge_index[0].astype(i32) + S_ROWS
    dst2 = mol2_edge_index[1].astype(i32)

    def mol_edges(src, dst):
        return jnp.concatenate([_pad_edges(src, 0), _pad_edges(dst, TRASH)],
                               axis=3)

    edges = jnp.stack([mol_edges(src1, dst1), mol_edges(src2, dst2)])

    ones_h = jnp.ones((CH, HW), jnp.float32)
    zeros_h = jnp.zeros((S_ROWS // NT, HW), jnp.float32)
    zeros128 = jnp.zeros((S_ROWS // NT, 128), jnp.float32)

    deg = _sc_degree(edges, ones_h, zeros_h).reshape(NC * S_ROWS, HW)

    pad_x = jnp.zeros((S_ROWS - N_NODES, 128), jnp.float32)
    x = jnp.concatenate([mol1_x, pad_x, mol2_x, pad_x], axis=0)
    g1 = _tc_layer1(x, deg, conv1_W)
    s1 = _msgpass128(g1, edges, zeros128)
    g2 = _tc_layer(s1, g1, _tile8(conv1_b), deg, conv2_W, 128)
    s2 = _msgpass128(g2, edges, zeros128)
    g3 = _tc_layer(s2, g2, _tile8(conv2_b), deg, conv3_W, 128)
    s3 = _msgpass128(g3, edges, zeros128)
    conv4_Wp = jnp.pad(conv4_W, ((0, 0), (0, 64)))
    g4 = _tc_layer(s3, g3, _tile8(conv3_b), deg, conv4_Wp, 128)
    s4 = _msgpass128(g4, edges, zeros128)

    pad_ids = jnp.full((S_ROWS - N_NODES,), N_GRAPHS + 1, i32)
    ids = jnp.concatenate([mol1_batch_ids.astype(i32), pad_ids,
                           mol2_batch_ids.astype(i32), pad_ids])
    ids3 = ids.reshape(NC * (S_ROWS // _PB), 1, _PB)
    pooled = _tc_pool(s4, g4, _tile8(jnp.pad(conv4_b, (0, 64))), deg, ids3)

    return _tc_classifier(
        pooled, mol1_notes, mol2_notes,
        fc1_W, _tile8(fc1_b), fc2_W, _tile8(fc2_b),
        cls1_W, _tile8(cls1_b), _tile8(bn1_g), _tile8(bn1_b),
        cls2_W, _tile8(cls2_b), _tile8(bn2_g), _tile8(bn2_b),
        cls3_W, _tile8(cls3_b))


# speedup vs baseline: 1.0243x; 1.0243x over previous
import functools

import jax
import jax.numpy as jnp
from jax import lax
from jax.experimental import pallas as pl
from jax.experimental.pallas import tpu as pltpu
from jax.experimental.pallas import tpu_sc as plsc

N_NODES = 10000
N_EDGES = 320000
N_GRAPHS = 256
NT = 16
NC = 2
CHUNK = 128
NCHUNKS = 158
NPAIR = NCHUNKS // 2
EPT = NCHUNKS * CHUNK
PAD = EPT * NT - N_EDGES
S_ROWS = 10240
TRASH = N_NODES
HW = 128


def _sc_mesh():
    return plsc.VectorSubcoreMesh(core_axis_name="c", subcore_axis_name="s",
                                  num_cores=NC, num_subcores=NT)


@functools.cache
def _make_degree():
    @functools.partial(
        pl.kernel,
        out_type=jax.ShapeDtypeStruct((NC, S_ROWS, HW), jnp.float32),
        mesh=_sc_mesh(),
        scratch_types=[
            pltpu.VMEM((NPAIR, 2, CHUNK), jnp.int32),
            pltpu.VMEM((CHUNK, HW), jnp.float32),
            pltpu.VMEM_SHARED((S_ROWS, HW), jnp.float32),
        ],
    )
    def degree(dst_hbm, ones_hbm, zeros_hbm, out_hbm, dst_v, ones_v, hist_sh):
        c = lax.axis_index("c")
        s = lax.axis_index("s")
        pltpu.sync_copy(ones_hbm, ones_v)
        pltpu.sync_copy(zeros_hbm,
                        hist_sh.at[pl.ds(s * (S_ROWS // NT), S_ROWS // NT)])
        pltpu.sync_copy(dst_hbm.at[c, s], dst_v)
        plsc.subcore_barrier()

        def body(j, carry):
            pltpu.sync_copy(ones_v, hist_sh.at[dst_v.at[j, 0]], add=True)
            pltpu.sync_copy(ones_v, hist_sh.at[dst_v.at[j, 1]], add=True)
            return carry

        lax.fori_loop(0, NPAIR, body, 0)
        plsc.subcore_barrier()
        rows = S_ROWS // NT
        pltpu.sync_copy(hist_sh.at[pl.ds(s * rows, rows)],
                        out_hbm.at[c, pl.ds(s * rows, rows)])

    return degree


def _sc_degree(dst_all, ones_h, zeros_h):
    return _make_degree()(dst_all, ones_h, zeros_h)


@functools.cache
def _make_msgpass(d):
    @functools.partial(
        pl.kernel,
        out_type=jax.ShapeDtypeStruct((NC * S_ROWS, d), jnp.float32),
        mesh=_sc_mesh(),
        scratch_types=[
            pltpu.VMEM((2, 2, CHUNK), jnp.int32),
            pltpu.VMEM((2, 2, CHUNK), jnp.int32),
            pltpu.VMEM((CHUNK, d), jnp.float32),
            pltpu.VMEM((CHUNK, d), jnp.float32),
            pltpu.VMEM_SHARED((S_ROWS, d), jnp.float32),
            pltpu.SemaphoreType.DMA,
            pltpu.SemaphoreType.DMA,
            pltpu.SemaphoreType.DMA,
            pltpu.SemaphoreType.DMA,
            pltpu.SemaphoreType.DMA,
        ],
    )
    def msgpass(g_hbm, src_hbm, dst_hbm, zeros_hbm, out_hbm,
                sidx, didx, rowa, rowb, s_sh,
                gsa, gsb, ssa, ssb, isem):
        c = lax.axis_index("c")
        s = lax.axis_index("s")
        stripe = S_ROWS // NT
        pltpu.sync_copy(zeros_hbm, s_sh.at[pl.ds(s * stripe, stripe)])
        plsc.subcore_barrier()

        def wait(sem, dst):
            pltpu.make_async_copy(g_hbm.at[pl.ds(0, dst.shape[0])]
                                  if dst.dtype == jnp.float32
                                  else src_hbm.at[0, 0, 0], dst, sem).wait()

        pltpu.async_copy(src_hbm.at[c, s, 0], sidx.at[0], isem)
        pltpu.async_copy(dst_hbm.at[c, s, 0], didx.at[0], isem)
        wait(isem, sidx.at[0])
        wait(isem, didx.at[0])
        pltpu.async_copy(g_hbm.at[sidx.at[0, 0]], rowa, gsa)
        pltpu.async_copy(g_hbm.at[sidx.at[0, 1]], rowb, gsb)
        pltpu.async_copy(src_hbm.at[c, s, 1], sidx.at[1], isem)
        pltpu.async_copy(dst_hbm.at[c, s, 1], didx.at[1], isem)

        def body(p, carry):
            bank = lax.rem(p, 2)
            nbank = 1 - bank
            wait(gsa, rowa)
            pltpu.async_copy(rowa, s_sh.at[didx.at[bank, 0]], ssa, add=True)
            wait(gsb, rowb)
            pltpu.async_copy(rowb, s_sh.at[didx.at[bank, 1]], ssb, add=True)
            wait(isem, sidx.at[0])
            wait(isem, didx.at[0])
            wait(ssa, rowa)
            pltpu.async_copy(g_hbm.at[sidx.at[nbank, 0]], rowa, gsa)
            wait(ssb, rowb)
            pltpu.async_copy(g_hbm.at[sidx.at[nbank, 1]], rowb, gsb)
            p2 = jnp.minimum(p + 2, NPAIR - 1)
            pltpu.async_copy(src_hbm.at[c, s, p2], sidx.at[bank], isem)
            pltpu.async_copy(dst_hbm.at[c, s, p2], didx.at[bank], isem)
            return carry

        lax.fori_loop(0, NPAIR, body, 0)
        wait(gsa, rowa)
        wait(gsb, rowb)
        wait(isem, sidx.at[0])
        wait(isem, didx.at[0])
        plsc.subcore_barrier()
        pltpu.sync_copy(s_sh.at[pl.ds(s * stripe, stripe)],
                        out_hbm.at[pl.ds(c * S_ROWS + s * stripe, stripe)])

    return msgpass


def _msgpass128(g, src_all, dst_all, zeros):
    return _make_msgpass(128)(g, src_all, dst_all, zeros)


_BM = 1024


def _tc_layer1(x, deg, w):
    def body(x_ref, deg_ref, w_ref, o_ref):
        dinv = lax.rsqrt(deg_ref[:, 0:1] + 1.0)
        o_ref[...] = dinv * jnp.dot(x_ref[...], w_ref[...],
                                    preferred_element_type=jnp.float32)

    m = NC * S_ROWS
    return pl.pallas_call(
        body,
        grid=(m // _BM,),
        in_specs=[
            pl.BlockSpec((_BM, 128), lambda i: (i, 0)),
            pl.BlockSpec((_BM, HW), lambda i: (i, 0)),
            pl.BlockSpec((128, 128), lambda i: (0, 0)),
        ],
        out_specs=pl.BlockSpec((_BM, 128), lambda i: (i, 0)),
        out_shape=jax.ShapeDtypeStruct((m, 128), jnp.float32),
    )(x, deg, w)


def _tc_layer(s_in, g_prev, b_prev, deg, w, d_out):
    def body(s_ref, g_ref, deg_ref, b_ref, w_ref, o_ref):
        dinv = lax.rsqrt(deg_ref[:, 0:1] + 1.0)
        x = jax.nn.relu(dinv * (s_ref[...] + g_ref[...]) + b_ref[0:1, :])
        o_ref[...] = dinv * jnp.dot(x, w_ref[...],
                                    preferred_element_type=jnp.float32)

    m, d_in = s_in.shape
    return pl.pallas_call(
        body,
        grid=(m // _BM,),
        in_specs=[
            pl.BlockSpec((_BM, d_in), lambda i: (i, 0)),
            pl.BlockSpec((_BM, d_in), lambda i: (i, 0)),
            pl.BlockSpec((_BM, HW), lambda i: (i, 0)),
            pl.BlockSpec((8, d_in), lambda i: (0, 0)),
            pl.BlockSpec((d_in, d_out), lambda i: (0, 0)),
        ],
        out_specs=pl.BlockSpec((_BM, d_out), lambda i: (i, 0)),
        out_shape=jax.ShapeDtypeStruct((m, d_out), jnp.float32),
    )(s_in, g_prev, deg, b_prev, w)


_PB = 2048


def _tc_pool(s4, g4, b4, deg, ids3):
    nb = S_ROWS // _PB

    def body(s_ref, g_ref, deg_ref, b_ref, ids_ref, o_ref, cnt_ref):
        j = pl.program_id(1)
        dinv = lax.rsqrt(deg_ref[:, 0:1] + 1.0)
        x = jax.nn.relu(dinv * (s_ref[...] + g_ref[...]) + b_ref[0:1, :])[:, :64]
        ids = ids_ref[0, 0, :]
        gid = lax.broadcasted_iota(jnp.int32, (N_GRAPHS, _PB), 0)
        p = (gid == ids[None, :]).astype(jnp.float32)
        ps = jnp.dot(p, x, preferred_element_type=jnp.float32)
        pc = jnp.sum(p, axis=1, keepdims=True)

        @pl.when(j == 0)
        def _():
            o_ref[0] = ps
            cnt_ref[...] = jnp.broadcast_to(pc, (N_GRAPHS, 128))

        @pl.when(j != 0)
        def _():
            o_ref[0] += ps
            cnt_ref[...] += jnp.broadcast_to(pc, (N_GRAPHS, 128))

        @pl.when(j == nb - 1)
        def _():
            o_ref[0] /= jnp.maximum(cnt_ref[:, 0:1], 1.0)

    return pl.pallas_call(
        body,
        grid=(NC, nb),
        in_specs=[
            pl.BlockSpec((_PB, 128), lambda m, j: (m * nb + j, 0)),
            pl.BlockSpec((_PB, 128), lambda m, j: (m * nb + j, 0)),
            pl.BlockSpec((_PB, HW), lambda m, j: (m * nb + j, 0)),
            pl.BlockSpec((8, 128), lambda m, j: (0, 0)),
            pl.BlockSpec((1, 1, _PB), lambda m, j: (m * nb + j, 0, 0)),
        ],
        out_specs=pl.BlockSpec((1, N_GRAPHS, 64), lambda m, j: (m, 0, 0)),
        out_shape=jax.ShapeDtypeStruct((NC, N_GRAPHS, 64), jnp.float32),
        scratch_shapes=[pltpu.VMEM((N_GRAPHS, 128), jnp.float32)],
    )(s4, g4, deg, b4, ids3)


def _tc_classifier(pooled, n1, n2, fc1_w, fc1_b, fc2_w, fc2_b,
                   cls1_w, cls1_b, bn1_g, bn1_b,
                   cls2_w, cls2_b, bn2_g, bn2_b, cls3_w, cls3_b):
    def bn(h, g_ref, b_ref):
        mu = jnp.mean(h, axis=0, keepdims=True)
        var = jnp.mean((h - mu) ** 2, axis=0, keepdims=True)
        return (h - mu) * lax.rsqrt(var + 1e-5) * g_ref[0:1, :] + b_ref[0:1, :]

    def body(pooled_ref, n1_ref, n2_ref, fc1w_ref, fc1b_ref, fc2w_ref,
             fc2b_ref, c1w_ref, c1b_ref, g1_ref, b1_ref, c2w_ref, c2b_ref,
             g2_ref, b2_ref, c3w_ref, c3b_ref, o_ref):
        def notes_mlp(r):
            t = jax.nn.relu(jnp.dot(r, fc1w_ref[...],
                                    preferred_element_type=jnp.float32)
                            + fc1b_ref[0:1, :])
            return jax.nn.relu(jnp.dot(t, fc2w_ref[...],
                                       preferred_element_type=jnp.float32)
                               + fc2b_ref[0:1, :])

        m1 = notes_mlp(n1_ref[...])
        m2 = notes_mlp(n2_ref[...])
        comb = jnp.concatenate(
            [pooled_ref[0], pooled_ref[1], m1, m2], axis=1)
        h = jax.nn.relu(jnp.dot(comb, c1w_ref[...],
                                preferred_element_type=jnp.float32)
                        + c1b_ref[0:1, :])
        h = bn(h, g1_ref, b1_ref)
        h = jax.nn.relu(jnp.dot(h, c2w_ref[...],
                                preferred_element_type=jnp.float32)
                        + c2b_ref[0:1, :])
        h = bn(h, g2_ref, b2_ref)
        o_ref[...] = (jnp.dot(h, c3w_ref[...],
                              preferred_element_type=jnp.float32)
                      + c3b_ref[0:1, :])

    return pl.pallas_call(
        body,
        out_shape=jax.ShapeDtypeStruct((N_GRAPHS, 128), jnp.float32),
    )(pooled, n1, n2, fc1_w, fc1_b, fc2_w, fc2_b, cls1_w, cls1_b,
      bn1_g, bn1_b, cls2_w, cls2_b, bn2_g, bn2_b, cls3_w, cls3_b)


def _pad_edges(a, fill):
    a = jnp.concatenate([a, jnp.full((PAD,), fill, jnp.int32)])
    return a.reshape(NT, NPAIR, 2, CHUNK)


def _tile8(b):
    return jnp.tile(b[None, :], (8, 1))


def kernel(mol1_x, mol1_edge_index, mol1_batch_ids, mol1_notes,
           mol2_x, mol2_edge_index, mol2_batch_ids, mol2_notes,
           conv1_W, conv1_b, conv2_W, conv2_b, conv3_W, conv3_b,
           conv4_W, conv4_b, fc1_W, fc1_b, fc2_W, fc2_b,
           cls1_W, cls1_b, bn1_g, bn1_b, cls2_W, cls2_b, bn2_g, bn2_b,
           cls3_W, cls3_b):
    i32 = jnp.int32
    src1 = mol1_edge_index[0].astype(i32)
    dst1 = mol1_edge_index[1].astype(i32)
    src2 = mol2_edge_index[0].astype(i32) + S_ROWS
    dst2 = mol2_edge_index[1].astype(i32)

    src_all = jnp.stack([_pad_edges(src1, 0), _pad_edges(src2, 0)])
    dst_all = jnp.stack([_pad_edges(dst1, TRASH), _pad_edges(dst2, TRASH)])

    ones_h = jnp.ones((CHUNK, HW), jnp.float32)
    zeros_h = jnp.zeros((S_ROWS // NT, HW), jnp.float32)
    zeros128 = jnp.zeros((S_ROWS // NT, 128), jnp.float32)

    deg = _sc_degree(dst_all, ones_h, zeros_h).reshape(NC * S_ROWS, HW)

    pad_x = jnp.zeros((S_ROWS - N_NODES, 128), jnp.float32)
    x = jnp.concatenate([mol1_x, pad_x, mol2_x, pad_x], axis=0)
    g1 = _tc_layer1(x, deg, conv1_W)
    s1 = _msgpass128(g1, src_all, dst_all, zeros128)
    g2 = _tc_layer(s1, g1, _tile8(conv1_b), deg, conv2_W, 128)
    s2 = _msgpass128(g2, src_all, dst_all, zeros128)
    g3 = _tc_layer(s2, g2, _tile8(conv2_b), deg, conv3_W, 128)
    s3 = _msgpass128(g3, src_all, dst_all, zeros128)
    conv4_Wp = jnp.pad(conv4_W, ((0, 0), (0, 64)))
    g4 = _tc_layer(s3, g3, _tile8(conv3_b), deg, conv4_Wp, 128)
    s4 = _msgpass128(g4, src_all, dst_all, zeros128)

    pad_ids = jnp.full((S_ROWS - N_NODES,), N_GRAPHS + 1, i32)
    ids = jnp.concatenate([mol1_batch_ids.astype(i32), pad_ids,
                           mol2_batch_ids.astype(i32), pad_ids])
    ids3 = ids.reshape(NC * (S_ROWS // _PB), 1, _PB)
    pooled = _tc_pool(s4, g4, _tile8(jnp.pad(conv4_b, (0, 64))), deg, ids3)

    return _tc_classifier(
        pooled, mol1_notes, mol2_notes,
        fc1_W, _tile8(fc1_b), fc2_W, _tile8(fc2_b),
        cls1_W, _tile8(cls1_b), _tile8(bn1_g), _tile8(bn1_b),
        cls2_W, _tile8(cls2_b), _tile8(bn2_g), _tile8(bn2_b),
        cls3_W, _tile8(cls3_b))
